# baseline (speedup 1.0000x reference)
import jax
import jax.numpy as jnp
from jax.experimental import pallas as pl
from jax.experimental.pallas import tpu as pltpu

B, E, K, D, C = 64, 64, 8, 1024, 100
E_BLK = 16
N_STEPS = E // E_BLK
TANH_FACTOR = 10.0


E_SUB = 8
N_SUB = E_BLK // E_SUB


def _ens_kernel(x_ref, keys_ref, ew_ref, eb_ref, vw_ref, vb_ref, tw_ref, tb_ref,
                ens_ref, tanh_ref, van_ref,
                w_ref, denom_ref, acc_ref):
    i = pl.program_id(0)
    h = pl.program_id(1)

    @pl.when(jnp.logical_and(i == 0, h == 0))
    def _routing():
        x = x_ref[...]
        norm = jnp.sqrt(jnp.sum(x * x, axis=1, keepdims=True))
        xn = x / jnp.maximum(norm, 1e-12)
        cos = jax.lax.dot_general(xn, keys_ref[...], (((1,), (1,)), ((), ())),
                                  preferred_element_type=jnp.float32)
        idxs = jax.lax.broadcasted_iota(jnp.int32, (B, E), 1)
        work = cos
        sel = jnp.zeros((B, E), dtype=jnp.bool_)
        sims = []
        for _ in range(K):
            m = jnp.max(work, axis=1, keepdims=True)
            is_max = work == m
            first_idx = jnp.min(jnp.where(is_max, idxs, E), axis=1, keepdims=True)
            first = idxs == first_idx
            sel = jnp.logical_or(sel, first)
            sims.append(m)
            work = jnp.where(first, -1e30, work)
        sel_f = sel.astype(jnp.float32)
        row = jax.lax.broadcasted_iota(jnp.int32, (E, E), 0)
        col = jax.lax.broadcasted_iota(jnp.int32, (E, E), 1)
        tri = (row < col).astype(jnp.float32)
        pos = jax.lax.dot_general(sel_f, tri, (((1,), (0,)), ((), ())),
                                  preferred_element_type=jnp.float32)
        w = jnp.zeros((B, E), dtype=jnp.float32)
        den = jnp.zeros((B, 1), dtype=jnp.float32)
        for k in range(K):
            w = jnp.where(jnp.logical_and(sel, pos == float(k)), sims[k], w)
            den = den + sims[k]
        w_ref[...] = w
        denom_ref[...] = den
        acc_ref[...] = jnp.zeros((B, C), jnp.float32)
        v = jax.lax.dot_general(x, vw_ref[...], (((1,), (1,)), ((), ())),
                                preferred_element_type=jnp.float32) + vb_ref[...]
        m2 = jnp.max(v, axis=1, keepdims=True)
        s = v - m2
        lse = jnp.log(jnp.sum(jnp.exp(s), axis=1, keepdims=True))
        van_ref[...] = s - lse
        th = jax.lax.dot_general(x, tw_ref[...], (((1,), (1,)), ((), ())),
                                 preferred_element_type=jnp.float32) + tb_ref[...]
        tanh_ref[...] = jnp.tanh(th * (1.0 / TANH_FACTOR)) * TANH_FACTOR

    x = x_ref[...]
    w = w_ref[...]
    idxs = jax.lax.broadcasted_iota(jnp.int32, (B, E), 1)
    acc = acc_ref[...]
    for j0 in range(E_SUB):
        j = h * E_SUB + j0
        e_idx = i * E_BLK + j
        wj = ew_ref[pl.ds(j, 1)][0]
        y = jax.lax.dot_general(x, wj, (((1,), (1,)), ((), ())),
                                preferred_element_type=jnp.float32)
        y = y + eb_ref[pl.ds(j, 1)]
        t = jnp.tanh(y * (1.0 / TANH_FACTOR)) * TANH_FACTOR
        wcol = jnp.sum(jnp.where(idxs == e_idx, w, 0.0), axis=1, keepdims=True)
        acc = acc + wcol * t
    acc_ref[...] = acc

    @pl.when(jnp.logical_and(i == N_STEPS - 1, h == N_SUB - 1))
    def _finish():
        ens_ref[...] = acc_ref[...] / denom_ref[...]


def _run(x, keys, expert_W, expert_b, vanilla_W, vb2, tanh_W, tb2):
    return pl.pallas_call(
        _ens_kernel,
        grid=(N_STEPS, N_SUB),
        in_specs=[
            pl.BlockSpec((B, D), lambda i, h: (0, 0)),
            pl.BlockSpec((E, D), lambda i, h: (0, 0)),
            pl.BlockSpec((E_BLK, C, D), lambda i, h: (i, 0, 0)),
            pl.BlockSpec((E_BLK, C), lambda i, h: (i, 0)),
            pl.BlockSpec((C, D), lambda i, h: (0, 0)),
            pl.BlockSpec((1, C), lambda i, h: (0, 0)),
            pl.BlockSpec((C, D), lambda i, h: (0, 0)),
            pl.BlockSpec((1, C), lambda i, h: (0, 0)),
        ],
        out_specs=[
            pl.BlockSpec((B, C), lambda i, h: (0, 0)),
            pl.BlockSpec((B, C), lambda i, h: (0, 0)),
            pl.BlockSpec((B, C), lambda i, h: (0, 0)),
        ],
        out_shape=[
            jax.ShapeDtypeStruct((B, C), jnp.float32),
            jax.ShapeDtypeStruct((B, C), jnp.float32),
            jax.ShapeDtypeStruct((B, C), jnp.float32),
        ],
        scratch_shapes=[
            pltpu.VMEM((B, E), jnp.float32),
            pltpu.VMEM((B, 1), jnp.float32),
            pltpu.VMEM((B, C), jnp.float32),
        ],
    )(x, keys, expert_W, expert_b, vanilla_W, vb2, tanh_W, tb2)


def kernel(x, keys, expert_W, expert_b, vanilla_W, vanilla_b, tanh_W, tanh_b,
           x_is_encoded=1):
    ens, tanh_out, van = _run(x, keys, expert_W, expert_b,
                              vanilla_W, vanilla_b.reshape(1, C),
                              tanh_W, tanh_b.reshape(1, C))
    return (ens, tanh_out, van)


# speedup vs baseline: 1.0748x; 1.0748x over previous
import jax
import jax.numpy as jnp
from jax.experimental import pallas as pl
from jax.experimental.pallas import tpu as pltpu

B, E, K, D, C = 64, 64, 8, 1024, 100
E_BLK = 16
N_STEPS = E // E_BLK
TANH_FACTOR = 10.0


def _ens_kernel(x_ref, keys_ref, ew_ref, eb_ref, vw_ref, vb_ref, tw_ref, tb_ref,
                ens_ref, tanh_ref, van_ref,
                w_ref, denom_ref, acc_ref):
    i = pl.program_id(0)

    @pl.when(i == 0)
    def _routing():
        x = x_ref[...]
        norm = jnp.sqrt(jnp.sum(x * x, axis=1, keepdims=True))
        xn = x / jnp.maximum(norm, 1e-12)
        cos = jax.lax.dot_general(xn, keys_ref[...], (((1,), (1,)), ((), ())),
                                  preferred_element_type=jnp.float32)
        idxs = jax.lax.broadcasted_iota(jnp.int32, (B, E), 1)
        work = cos
        sel = jnp.zeros((B, E), dtype=jnp.bool_)
        sims = []
        for _ in range(K):
            m = jnp.max(work, axis=1, keepdims=True)
            is_max = work == m
            first_idx = jnp.min(jnp.where(is_max, idxs, E), axis=1, keepdims=True)
            first = idxs == first_idx
            sel = jnp.logical_or(sel, first)
            sims.append(m)
            work = jnp.where(first, -1e30, work)
        sel_f = sel.astype(jnp.float32)
        row = jax.lax.broadcasted_iota(jnp.int32, (E, E), 0)
        col = jax.lax.broadcasted_iota(jnp.int32, (E, E), 1)
        tri = (row < col).astype(jnp.float32)
        pos = jax.lax.dot_general(sel_f, tri, (((1,), (0,)), ((), ())),
                                  preferred_element_type=jnp.float32)
        w = jnp.zeros((B, E), dtype=jnp.float32)
        den = jnp.zeros((B, 1), dtype=jnp.float32)
        for k in range(K):
            w = jnp.where(jnp.logical_and(sel, pos == float(k)), sims[k], w)
            den = den + sims[k]
        w_ref[...] = w
        denom_ref[...] = den
        acc_ref[...] = jnp.zeros((B, C), jnp.float32)
        v = jax.lax.dot_general(x, vw_ref[...], (((1,), (1,)), ((), ())),
                                preferred_element_type=jnp.float32) + vb_ref[...]
        m2 = jnp.max(v, axis=1, keepdims=True)
        s = v - m2
        lse = jnp.log(jnp.sum(jnp.exp(s), axis=1, keepdims=True))
        van_ref[...] = s - lse
        th = jax.lax.dot_general(x, tw_ref[...], (((1,), (1,)), ((), ())),
                                 preferred_element_type=jnp.float32) + tb_ref[...]
        tanh_ref[...] = jnp.tanh(th * (1.0 / TANH_FACTOR)) * TANH_FACTOR

    x = x_ref[...]
    w = w_ref[...]
    idxs = jax.lax.broadcasted_iota(jnp.int32, (B, E), 1)
    acc = acc_ref[...]
    for j in range(E_BLK):
        e_idx = i * E_BLK + j
        wj = ew_ref[j]
        y = jax.lax.dot_general(x, wj, (((1,), (1,)), ((), ())),
                                preferred_element_type=jnp.float32)
        y = y + eb_ref[j][None, :]
        t = jnp.tanh(y * (1.0 / TANH_FACTOR)) * TANH_FACTOR
        wcol = jnp.sum(jnp.where(idxs == e_idx, w, 0.0), axis=1, keepdims=True)
        acc = acc + wcol * t
    acc_ref[...] = acc

    @pl.when(i == N_STEPS - 1)
    def _finish():
        ens_ref[...] = acc_ref[...] / denom_ref[...]


def _run(x, keys, expert_W, expert_b, vanilla_W, vb2, tanh_W, tb2):
    return pl.pallas_call(
        _ens_kernel,
        grid=(N_STEPS,),
        in_specs=[
            pl.BlockSpec((B, D), lambda i: (0, 0)),
            pl.BlockSpec((E, D), lambda i: (0, 0)),
            pl.BlockSpec((E_BLK, C, D), lambda i: (i, 0, 0)),
            pl.BlockSpec((E_BLK, C), lambda i: (i, 0)),
            pl.BlockSpec((C, D), lambda i: (0, 0)),
            pl.BlockSpec((1, C), lambda i: (0, 0)),
            pl.BlockSpec((C, D), lambda i: (0, 0)),
            pl.BlockSpec((1, C), lambda i: (0, 0)),
        ],
        out_specs=[
            pl.BlockSpec((B, C), lambda i: (0, 0)),
            pl.BlockSpec((B, C), lambda i: (0, 0)),
            pl.BlockSpec((B, C), lambda i: (0, 0)),
        ],
        out_shape=[
            jax.ShapeDtypeStruct((B, C), jnp.float32),
            jax.ShapeDtypeStruct((B, C), jnp.float32),
            jax.ShapeDtypeStruct((B, C), jnp.float32),
        ],
        scratch_shapes=[
            pltpu.VMEM((B, E), jnp.float32),
            pltpu.VMEM((B, 1), jnp.float32),
            pltpu.VMEM((B, C), jnp.float32),
        ],
    )(x, keys, expert_W, expert_b, vanilla_W, vb2, tanh_W, tb2)


def kernel(x, keys, expert_W, expert_b, vanilla_W, vanilla_b, tanh_W, tanh_b,
           x_is_encoded=1):
    ens, tanh_out, van = _run(x, keys, expert_W, expert_b,
                              vanilla_W, vanilla_b.reshape(1, C),
                              tanh_W, tanh_b.reshape(1, C))
    return (ens, tanh_out, van)
